# baseline (speedup 1.0000x reference)
import functools

import jax
import jax.numpy as jnp
from jax import lax
from jax.experimental import pallas as pl
from jax.experimental.pallas import tpu as pltpu
from jax.experimental.pallas import tpu_sc as plsc

_V = 1000000
_D = 20
_DP = 32
_B = 16384
_P = 30
_N = _B * _P

_NC, _NS = 2, 16
_NW = _NC * _NS

_INV15 = -286331153
_INV5 = -858993459

_mesh = plsc.VectorSubcoreMesh(core_axis_name="c", subcore_axis_name="s")
_CP = pltpu.CompilerParams(use_tc_tiling_on_sc=False, needs_layout_passes=False)

_A_ROWS = _V // _NW
_A_CH = 625
_A_NCH = _A_ROWS // _A_CH
_A_BUF = _A_CH + 3


def _widen_tec(b20, b32):
    i16 = lax.iota(jnp.int32, 16)

    def rowstep(r, carry):
        b32[r, pl.ds(0, 16)] = b20[r, pl.ds(0, 16)]
        return carry

    lax.fori_loop(0, _A_CH, rowstep, 0)

    def quadstep(t, carry):
        f = i16 + 16 * t
        jj = jnp.bitwise_and(f, 3) + 16
        rw = f >> 2
        v = plsc.load_gather(b20, [rw, jj - 16 + 16])
        plsc.store_scatter(b32, [rw, jj], v)
        return carry

    lax.fori_loop(0, (_A_CH * 4 + 15) // 16, quadstep, 0)


@functools.partial(
    pl.kernel,
    mesh=_mesh,
    out_type=jax.ShapeDtypeStruct((_V, _DP), jnp.float32),
    scratch_types=[
        pltpu.VMEM((_A_BUF, _D), jnp.float32),
        pltpu.VMEM((_A_BUF, _D), jnp.float32),
        pltpu.VMEM((_A_BUF, _DP), jnp.float32),
        pltpu.VMEM((_A_BUF, _DP), jnp.float32),
        pltpu.SemaphoreType.DMA,
        pltpu.SemaphoreType.DMA,
        pltpu.SemaphoreType.DMA,
        pltpu.SemaphoreType.DMA,
    ],
    compiler_params=_CP,
)
def _widen(src_hbm, dst_hbm, b20a, b20b, b32a, b32b, l0, l1, s0, s1):
    wid = lax.axis_index("s") * _NC + lax.axis_index("c")
    base = wid * _A_ROWS
    b20 = (b20a, b20b)
    b32 = (b32a, b32b)
    lsems = (l0, l1)
    ssems = (s0, s1)
    ld = [None, None]
    st = [None, None]
    for c in range(_A_NCH + 1):
        b = c % 2
        if c < _A_NCH:
            if c >= 2:
                st[b].wait()
            ld[b] = pltpu.async_copy(
                src_hbm.at[pl.ds(base + c * _A_CH, _A_CH)],
                b20[b].at[pl.ds(0, _A_CH)],
                lsems[b],
            )
        if c >= 1:
            pb = 1 - b
            ld[pb].wait()
            _widen_tec(b20[pb], b32[pb])
            st[pb] = pltpu.async_copy(
                b32[pb].at[pl.ds(0, _A_CH)],
                dst_hbm.at[pl.ds(base + (c - 1) * _A_CH, _A_CH)],
                ssems[pb],
            )
    st[(_A_NCH - 1) % 2].wait()
    st[_A_NCH % 2].wait()


_B_ROWS = _B // _NW
_B_RW = 32
_B_CI = _B_RW * _P
_B_NCH = _B_ROWS // _B_RW


@functools.partial(
    pl.kernel,
    mesh=_mesh,
    out_type=jax.ShapeDtypeStruct((_N, _D), jnp.float32),
    scratch_types=[
        pltpu.VMEM((_B_ROWS, _P), jnp.int32),
        pltpu.VMEM((_B_CI,), jnp.int32),
        pltpu.VMEM((_B_CI,), jnp.int32),
        pltpu.VMEM((_B_CI, _DP), jnp.float32),
        pltpu.VMEM((_B_CI, _DP), jnp.float32),
        pltpu.VMEM((_B_CI, _D), jnp.float32),
        pltpu.VMEM((_B_CI, _D), jnp.float32),
        pltpu.VMEM((80,), jnp.int32),
        pltpu.VMEM((80,), jnp.int32),
        pltpu.SemaphoreType.DMA,
        pltpu.SemaphoreType.DMA,
        pltpu.SemaphoreType.DMA,
        pltpu.SemaphoreType.DMA,
    ],
    compiler_params=_CP,
)
def _gather(
    table_hbm, x_hbm, out_hbm,
    xv, ci0, ci1, r320, r321, r200, r201, patj, patr,
    g0, g1, w0, w1,
):
    wid = lax.axis_index("s") * _NC + lax.axis_index("c")
    base = wid * _B_ROWS * _P
    pltpu.sync_copy(x_hbm.at[pl.ds(wid * _B_ROWS, _B_ROWS)], xv)

    i16 = lax.iota(jnp.int32, 16)
    for v in range(5):
        o = i16 + 16 * v
        j = o % 20
        patj[pl.ds(16 * v, 16)] = j
        patr[pl.ds(16 * v, 16)] = ((o - j) >> 2) * _INV5

    def flatten(c, ci):
        rb = c * _B_RW

        def fstep(k, carry):
            o = i16 + 16 * k
            m = o % 30
            rl = ((o - m) >> 1) * _INV15
            ci[pl.ds(16 * k, 16)] = plsc.load_gather(xv, [rl + rb, m])
            return carry

        lax.fori_loop(0, _B_CI // 16, fstep, 0)

    def compact(b32, b20):
        def gstep(g, carry):
            for v in range(5):
                j = patj[pl.ds(16 * v, 16)]
                rr = patr[pl.ds(16 * v, 16)] + 4 * g
                plsc.store_scatter(b20, [rr, j], plsc.load_gather(b32, [rr, j]))
            return carry

        lax.fori_loop(0, _B_CI // 4, gstep, 0)

    cis = (ci0, ci1)
    b32s = (r320, r321)
    b20s = (r200, r201)
    gsems = (g0, g1)
    wsems = (w0, w1)
    g = [None, None]
    wb = [None, None]
    for c in range(_B_NCH + 1):
        b = c % 2
        if c < _B_NCH:
            flatten(c, cis[b])
            g[b] = pltpu.async_copy(table_hbm.at[cis[b]], b32s[b], gsems[b])
        if c >= 1:
            pb = 1 - b
            g[pb].wait()
            if c >= 3:
                wb[pb].wait()
            compact(b32s[pb], b20s[pb])
            wb[pb] = pltpu.async_copy(
                b20s[pb],
                out_hbm.at[pl.ds(base + (c - 1) * _B_CI, _B_CI)],
                wsems[pb],
            )
    wb[(_B_NCH - 1) % 2].wait()
    wb[_B_NCH % 2].wait()


def kernel(x, table):
    table32 = _widen(table)
    out = _gather(table32, x.astype(jnp.int32))
    return out.reshape(_B, _P, _D)


# speedup vs baseline: 1.3824x; 1.3824x over previous
import functools

import jax
import jax.numpy as jnp
from jax import lax
from jax.experimental import pallas as pl
from jax.experimental.pallas import tpu as pltpu
from jax.experimental.pallas import tpu_sc as plsc

_V = 1000000
_D = 20
_DP = 32
_B = 16384
_P = 30
_N = _B * _P

_NC, _NS = 2, 16
_NW = _NC * _NS

_INV15 = -286331153

_mesh = plsc.VectorSubcoreMesh(core_axis_name="c", subcore_axis_name="s")
_CP = pltpu.CompilerParams(use_tc_tiling_on_sc=False, needs_layout_passes=False)

_B_ROWS = _B // _NW
_B_RW = 32
_B_CI = _B_RW * _P
_B_NCH = _B_ROWS // _B_RW


@functools.partial(
    pl.kernel,
    mesh=_mesh,
    out_type=jax.ShapeDtypeStruct((_B, _P, _D), jnp.float32),
    scratch_types=[
        pltpu.VMEM((_B_ROWS, _P), jnp.int32),
        pltpu.VMEM((_B_CI,), jnp.int32),
        pltpu.VMEM((_B_CI,), jnp.int32),
        pltpu.VMEM((_B_CI, _DP), jnp.float32),
        pltpu.VMEM((_B_CI, _DP), jnp.float32),
        pltpu.VMEM((_B_RW, _P, _D), jnp.float32),
        pltpu.VMEM((_B_RW, _P, _D), jnp.float32),
        pltpu.SemaphoreType.DMA,
        pltpu.SemaphoreType.DMA,
        pltpu.SemaphoreType.DMA,
        pltpu.SemaphoreType.DMA,
    ],
    compiler_params=_CP,
)
def _gather(
    table_hbm, x_hbm, out_hbm,
    xv, ci0, ci1, r320, r321, b30, b31,
    g0, g1, w0, w1,
):
    wid = lax.axis_index("s") * _NC + lax.axis_index("c")
    pltpu.sync_copy(x_hbm.at[pl.ds(wid * _B_ROWS, _B_ROWS)], xv)

    i16 = lax.iota(jnp.int32, 16)
    tail_cols = i16 + 16
    tail_mask = i16 < (_D - 16)

    def flatten(c, ci):
        rb = c * _B_RW

        def fstep(k, carry):
            o = i16 + 16 * k
            m = o % 30
            rl = ((o - m) >> 1) * _INV15
            ci[pl.ds(16 * k, 16)] = plsc.load_gather(xv, [rl + rb, m])
            return carry

        lax.fori_loop(0, _B_CI // 16, fstep, 0)

    def compact(b32, b3):
        def brstep(br, carry):
            def pstep(p, carry2):
                r = br * _P + p
                b3[br, p, pl.ds(0, 16)] = b32[r, pl.ds(0, 16)]
                v1 = b32[r, pl.ds(16, 16)]
                plsc.store_scatter(
                    b3, [br + 0 * i16, p + 0 * i16, tail_cols], v1, mask=tail_mask
                )
                return carry2

            lax.fori_loop(0, _P, pstep, carry)
            return carry

        lax.fori_loop(0, _B_RW, brstep, 0)

    cis = (ci0, ci1)
    b32s = (r320, r321)
    b3s = (b30, b31)
    gsems = (g0, g1)
    wsems = (w0, w1)
    g = [None, None]
    wb = [None, None]
    for c in range(_B_NCH + 1):
        b = c % 2
        if c < _B_NCH:
            flatten(c, cis[b])
            g[b] = pltpu.async_copy(table_hbm.at[cis[b]], b32s[b], gsems[b])
        if c >= 1:
            pb = 1 - b
            g[pb].wait()
            if c >= 3:
                wb[pb].wait()
            compact(b32s[pb], b3s[pb])
            wb[pb] = pltpu.async_copy(
                b3s[pb],
                out_hbm.at[pl.ds(wid * _B_ROWS + (c - 1) * _B_RW, _B_RW)],
                wsems[pb],
            )
    wb[(_B_NCH - 1) % 2].wait()
    wb[_B_NCH % 2].wait()


def kernel(x, table):
    table32 = jnp.pad(table, ((0, 0), (0, _DP - _D)))
    return _gather(table32, x.astype(jnp.int32))
